# Initial kernel scaffold; baseline (speedup 1.0000x reference)
import jax
import jax.numpy as jnp
from jax.experimental import pallas as pl


def kernel(heatmap):
    raise NotImplementedError("write your pallas kernel here")


# speedup vs baseline: 5.7258x; 5.7258x over previous
import functools

import jax
import jax.numpy as jnp
from jax import lax
from jax.experimental import pallas as pl
from jax.experimental.pallas import tpu as pltpu
from jax.experimental.pallas import tpu_sc as plsc

B = 8
H = 224
NR = 217
RAD = 3
NT = 10
NEG = -1e30
BIG = 9999


def _tree7(xs):
    a = (xs[0] + xs[1]) + (xs[2] + xs[3])
    b = (xs[4] + xs[5]) + xs[6]
    return a + b


def _xlane_max(v):
    w = jnp.maximum(v, lax.rev(v, (0,)))
    m = w[0]
    for l in range(1, 8):
        m = jnp.maximum(m, w[l])
    return m


def _xlane_min(v):
    w = jnp.minimum(v, lax.rev(v, (0,)))
    m = w[0]
    for l in range(1, 8):
        m = jnp.minimum(m, w[l])
    return m


def _store_scalar(ref, i, val, iota):
    base = pl.multiple_of((i // 16) * 16, 16)
    w = ref[pl.ds(base, 16)]
    ref[pl.ds(base, 16)] = jnp.where(base + iota == i, val, w)


@functools.partial(
    pl.kernel,
    out_type=jax.ShapeDtypeStruct((B, 32), jnp.int32),
    mesh=plsc.VectorSubcoreMesh(core_axis_name="c", subcore_axis_name="s"),
    scratch_types=[
        pltpu.VMEM((H, H), jnp.float32),
        pltpu.VMEM((NR, 240), jnp.float32),
        pltpu.VMEM((240,), jnp.float32),
        pltpu.VMEM((H,), jnp.float32),
        pltpu.VMEM((32,), jnp.int32),
        pltpu.VMEM((48,), jnp.float32),
    ],
)
def _sampler(hm_hbm, out_hbm, hm_v, s_v, vrow_v, rowmax_v, coords_v, vtmp_v):
    cid = lax.axis_index("c")
    sid = lax.axis_index("s")
    wid = sid * 2 + cid
    iota = lax.iota(jnp.int32, 16)

    @pl.when(wid < B)
    def _():
        b = wid
        pltpu.sync_copy(hm_hbm.at[b], hm_v)
        vrow_v[pl.ds(224, 16)] = jnp.zeros((16,), jnp.float32)
        rowmax_v[pl.ds(208, 16)] = jnp.full((16,), NEG, jnp.float32)

        def init_row(r, carry):
            for j in range(14):
                v = _tree7([hm_v[r + dr, pl.ds(16 * j, 16)] for dr in range(7)])
                vrow_v[pl.ds(16 * j, 16)] = v
            macc = jnp.full((16,), NEG, jnp.float32)
            for j in range(14):
                sj = _tree7([vrow_v[pl.ds(16 * j + dc, 16)] for dc in range(7)])
                if j == 13:
                    sj = jnp.where(iota + 208 <= 216, sj, NEG)
                s_v[r, pl.ds(16 * j, 16)] = sj
                macc = jnp.maximum(macc, sj)
            _store_scalar(rowmax_v, r, _xlane_max(macc), iota)
            return carry

        lax.fori_loop(0, NR, init_row, 0)

        def step(t, carry):
            macc = jnp.full((16,), NEG, jnp.float32)
            for j in range(14):
                macc = jnp.maximum(macc, rowmax_v[pl.ds(16 * j, 16)])
            gm = _xlane_max(macc)
            cand = jnp.full((16,), BIG, jnp.int32)
            for j in range(14):
                v = rowmax_v[pl.ds(16 * j, 16)]
                cand = jnp.minimum(cand, jnp.where(v == gm, iota + 16 * j, BIG))
            r_star = _xlane_min(cand)
            cand = jnp.full((16,), BIG, jnp.int32)
            for j in range(14):
                v = s_v[r_star, pl.ds(16 * j, 16)]
                cand = jnp.minimum(cand, jnp.where(v == gm, iota + 16 * j, BIG))
            c_star = _xlane_min(cand)
            base = pl.multiple_of((t // 8) * 16, 16)
            idx0 = 2 * t - base
            w = coords_v[pl.ds(base, 16)]
            w = jnp.where(iota == idx0, c_star + RAD, w)
            w = jnp.where(iota == idx0 + 1, r_star + RAD, w)
            coords_v[pl.ds(base, 16)] = w
            ca = pl.multiple_of((c_star // 16) * 16, 16)
            ca1 = pl.multiple_of(ca + 16, 16)
            in0 = (ca + iota >= c_star) & (ca + iota <= c_star + 6)
            in1 = (ca1 + iota >= c_star) & (ca1 + iota <= c_star + 6)
            for dr in range(7):
                row = r_star + dr
                w0 = hm_v[row, pl.ds(ca, 16)]
                hm_v[row, pl.ds(ca, 16)] = jnp.where(in0, jnp.float32(0.0), w0)
                w1 = hm_v[row, pl.ds(ca1, 16)]
                hm_v[row, pl.ds(ca1, 16)] = jnp.where(in1, jnp.float32(0.0), w1)
            cg = pl.multiple_of((jnp.maximum(c_star - 6, 0) // 16) * 16, 16)
            cg1 = pl.multiple_of(cg + 16, 16)
            cg2 = pl.multiple_of(cg + 32, 16)
            for d in range(13):
                rr = r_star - 6 + d

                @pl.when((rr >= 0) & (rr <= NR - 1))
                def _():
                    for half, off in enumerate((cg, cg1, cg2)):
                        v = _tree7([hm_v[rr + dr, pl.ds(off, 16)]
                                    for dr in range(7)])
                        vtmp_v[pl.ds(16 * half, 16)] = v
                    for half, off in enumerate((cg, cg1)):
                        snew = _tree7([vtmp_v[pl.ds(16 * half + dc, 16)]
                                       for dc in range(7)])
                        snew = jnp.where(off + iota <= 216, snew, NEG)
                        s_v[rr, pl.ds(off, 16)] = snew
                    macc2 = jnp.full((16,), NEG, jnp.float32)
                    for j in range(14):
                        macc2 = jnp.maximum(macc2, s_v[rr, pl.ds(16 * j, 16)])
                    _store_scalar(rowmax_v, rr, _xlane_max(macc2), iota)

            return carry

        lax.fori_loop(0, NT, step, 0)
        pltpu.sync_copy(coords_v, out_hbm.at[b])


def kernel(heatmap):
    hm3 = heatmap.reshape(B, H, H)
    out = _sampler(hm3)
    return out.reshape(B, 16, 2)[:, :NT, :].astype(jnp.int64)


# speedup vs baseline: 6.1398x; 1.0723x over previous
import functools

import jax
import jax.numpy as jnp
from jax import lax
from jax.experimental import pallas as pl
from jax.experimental.pallas import tpu as pltpu
from jax.experimental.pallas import tpu_sc as plsc

B = 8
H = 224
NR = 217
RAD = 3
NT = 10
NEG = -1e30
BIG = 9999


def _tree7(xs):
    a = (xs[0] + xs[1]) + (xs[2] + xs[3])
    b = (xs[4] + xs[5]) + xs[6]
    return a + b


def _xlane_max(v):
    w = jnp.maximum(v, lax.rev(v, (0,)))
    m = w[0]
    for l in range(1, 8):
        m = jnp.maximum(m, w[l])
    return m


def _xlane_min(v):
    w = jnp.minimum(v, lax.rev(v, (0,)))
    m = w[0]
    for l in range(1, 8):
        m = jnp.minimum(m, w[l])
    return m


def _store_scalar(ref, i, val, iota):
    base = pl.multiple_of((i // 16) * 16, 16)
    w = ref[pl.ds(base, 16)]
    ref[pl.ds(base, 16)] = jnp.where(base + iota == i, val, w)


@functools.partial(
    pl.kernel,
    out_type=jax.ShapeDtypeStruct((B, 32), jnp.int32),
    mesh=plsc.VectorSubcoreMesh(core_axis_name="c", subcore_axis_name="s"),
    scratch_types=[
        pltpu.VMEM((H, H), jnp.float32),
        pltpu.VMEM((NR, 240), jnp.float32),
        pltpu.VMEM((4, 240), jnp.float32),
        pltpu.VMEM((H,), jnp.float32),
        pltpu.VMEM((32,), jnp.int32),
        pltpu.VMEM((48,), jnp.float32),
    ],
)
def _sampler(hm_hbm, out_hbm, hm_v, s_v, vrow_v, rowmax_v, coords_v, vtmp_v):
    cid = lax.axis_index("c")
    sid = lax.axis_index("s")
    wid = sid * 2 + cid
    iota = lax.iota(jnp.int32, 16)

    @pl.when(wid < B)
    def _():
        b = wid
        pltpu.sync_copy(hm_hbm.at[b], hm_v)
        for q in range(4):
            vrow_v[q, pl.ds(224, 16)] = jnp.zeros((16,), jnp.float32)
        rowmax_v[pl.ds(208, 16)] = jnp.full((16,), NEG, jnp.float32)

        def _horizontal(q, r):
            macc = jnp.full((16,), NEG, jnp.float32)
            for j in range(14):
                sj = _tree7([vrow_v[q, pl.ds(16 * j + dc, 16)]
                             for dc in range(7)])
                if j == 13:
                    sj = jnp.where(iota + 208 <= 216, sj, NEG)
                s_v[r, pl.ds(16 * j, 16)] = sj
                macc = jnp.maximum(macc, sj)
            _store_scalar(rowmax_v, r, _xlane_max(macc), iota)

        def block4(i, carry):
            rbase = 4 * i
            for j in range(14):
                h = [hm_v[rbase + dr, pl.ds(16 * j, 16)] for dr in range(10)]
                for q in range(4):
                    vrow_v[q, pl.ds(16 * j, 16)] = _tree7(h[q:q + 7])
            for q in range(4):
                _horizontal(q, rbase + q)
            return carry

        def single(r, carry):
            for j in range(14):
                v = _tree7([hm_v[r + dr, pl.ds(16 * j, 16)] for dr in range(7)])
                vrow_v[0, pl.ds(16 * j, 16)] = v
            _horizontal(0, r)
            return carry

        lax.fori_loop(0, NR // 4, block4, 0)
        lax.fori_loop(4 * (NR // 4), NR, single, 0)

        def step(t, carry):
            macc = jnp.full((16,), NEG, jnp.float32)
            for j in range(14):
                macc = jnp.maximum(macc, rowmax_v[pl.ds(16 * j, 16)])
            gm = _xlane_max(macc)
            cand = jnp.full((16,), BIG, jnp.int32)
            for j in range(14):
                v = rowmax_v[pl.ds(16 * j, 16)]
                cand = jnp.minimum(cand, jnp.where(v == gm, iota + 16 * j, BIG))
            r_star = _xlane_min(cand)
            cand = jnp.full((16,), BIG, jnp.int32)
            for j in range(14):
                v = s_v[r_star, pl.ds(16 * j, 16)]
                cand = jnp.minimum(cand, jnp.where(v == gm, iota + 16 * j, BIG))
            c_star = _xlane_min(cand)
            base = pl.multiple_of((t // 8) * 16, 16)
            idx0 = 2 * t - base
            w = coords_v[pl.ds(base, 16)]
            w = jnp.where(iota == idx0, c_star + RAD, w)
            w = jnp.where(iota == idx0 + 1, r_star + RAD, w)
            coords_v[pl.ds(base, 16)] = w
            ca = pl.multiple_of((c_star // 16) * 16, 16)
            ca1 = pl.multiple_of(ca + 16, 16)
            in0 = (ca + iota >= c_star) & (ca + iota <= c_star + 6)
            in1 = (ca1 + iota >= c_star) & (ca1 + iota <= c_star + 6)
            for dr in range(7):
                row = r_star + dr
                w0 = hm_v[row, pl.ds(ca, 16)]
                hm_v[row, pl.ds(ca, 16)] = jnp.where(in0, jnp.float32(0.0), w0)
                w1 = hm_v[row, pl.ds(ca1, 16)]
                hm_v[row, pl.ds(ca1, 16)] = jnp.where(in1, jnp.float32(0.0), w1)
            cg = pl.multiple_of((jnp.maximum(c_star - 6, 0) // 16) * 16, 16)
            cg1 = pl.multiple_of(cg + 16, 16)
            cg2 = pl.multiple_of(cg + 32, 16)
            for d in range(13):
                rr = r_star - 6 + d

                @pl.when((rr >= 0) & (rr <= NR - 1))
                def _():
                    for half, off in enumerate((cg, cg1, cg2)):
                        v = _tree7([hm_v[rr + dr, pl.ds(off, 16)]
                                    for dr in range(7)])
                        vtmp_v[pl.ds(16 * half, 16)] = v
                    for half, off in enumerate((cg, cg1)):
                        snew = _tree7([vtmp_v[pl.ds(16 * half + dc, 16)]
                                       for dc in range(7)])
                        snew = jnp.where(off + iota <= 216, snew, NEG)
                        s_v[rr, pl.ds(off, 16)] = snew
                    macc2 = jnp.full((16,), NEG, jnp.float32)
                    for j in range(14):
                        macc2 = jnp.maximum(macc2, s_v[rr, pl.ds(16 * j, 16)])
                    _store_scalar(rowmax_v, rr, _xlane_max(macc2), iota)

            return carry

        lax.fori_loop(0, NT, step, 0)
        pltpu.sync_copy(coords_v, out_hbm.at[b])


def kernel(heatmap):
    hm3 = heatmap.reshape(B, H, H)
    out = _sampler(hm3)
    return out.reshape(B, 16, 2)[:, :NT, :].astype(jnp.int64)


# speedup vs baseline: 6.7263x; 1.0955x over previous
import functools

import jax
import jax.numpy as jnp
from jax import lax
from jax.experimental import pallas as pl
from jax.experimental.pallas import tpu as pltpu
from jax.experimental.pallas import tpu_sc as plsc

B = 8
H = 224
NR = 217
RAD = 3
NT = 10
NEG = -1e30
BIG = 9999
CHUNK = 64


def _tree7(xs):
    a = (xs[0] + xs[1]) + (xs[2] + xs[3])
    b = (xs[4] + xs[5]) + xs[6]
    return a + b


def _xlane_max(v):
    w = jnp.maximum(v, lax.rev(v, (0,)))
    m = w[0]
    for l in range(1, 8):
        m = jnp.maximum(m, w[l])
    return m


def _xlane_min(v):
    w = jnp.minimum(v, lax.rev(v, (0,)))
    m = w[0]
    for l in range(1, 8):
        m = jnp.minimum(m, w[l])
    return m


def _store_scalar(ref, i, val, iota):
    base = pl.multiple_of((i // 16) * 16, 16)
    w = ref[pl.ds(base, 16)]
    ref[pl.ds(base, 16)] = jnp.where(base + iota == i, val, w)


@functools.partial(
    pl.kernel,
    out_type=(jax.ShapeDtypeStruct((B, H, H), jnp.float32),
              jax.ShapeDtypeStruct((B, 256), jnp.float32)),
    mesh=plsc.VectorSubcoreMesh(core_axis_name="c", subcore_axis_name="s"),
    scratch_types=[
        pltpu.VMEM((232, H), jnp.float32),
        pltpu.VMEM((CHUNK, H), jnp.float32),
        pltpu.VMEM((4, 240), jnp.float32),
        pltpu.VMEM((CHUNK,), jnp.float32),
    ],
)
def _build(hm_hbm, s_hbm, rm_hbm, hm_v, s_v, vrow_v, rowmax_v):
    cid = lax.axis_index("c")
    sid = lax.axis_index("s")
    img = sid // 4
    role = sid % 4
    b = cid * 4 + img
    iota = lax.iota(jnp.int32, 16)
    r0 = CHUNK * role

    for k in range(3):
        @pl.when(role == k)
        def _():
            pltpu.sync_copy(hm_hbm.at[b, pl.ds(CHUNK * k, 72)],
                            hm_v.at[pl.ds(CHUNK * k, 72)])

    @pl.when(role == 3)
    def _():
        pltpu.sync_copy(hm_hbm.at[b, pl.ds(192, 32)],
                        hm_v.at[pl.ds(192, 32)])

    for q in range(4):
        vrow_v[q, pl.ds(224, 16)] = jnp.zeros((16,), jnp.float32)

    def _horizontal(q, lr, gr):
        macc = jnp.full((16,), NEG, jnp.float32)
        for j in range(14):
            sj = _tree7([vrow_v[q, pl.ds(16 * j + dc, 16)]
                         for dc in range(7)])
            if j == 13:
                sj = jnp.where(iota + 208 <= 216, sj, NEG)
            s_v[lr, pl.ds(16 * j, 16)] = sj
            macc = jnp.maximum(macc, sj)
        val = jnp.where(gr <= NR - 1, _xlane_max(macc), NEG)
        _store_scalar(rowmax_v, lr, val, iota)

    def block4(i, carry):
        gbase = r0 + 4 * i

        @pl.when(gbase < H)
        def _():
            for j in range(14):
                h = [hm_v[gbase + dr, pl.ds(16 * j, 16)] for dr in range(10)]
                for q in range(4):
                    vrow_v[q, pl.ds(16 * j, 16)] = _tree7(h[q:q + 7])
            for q in range(4):
                _horizontal(q, 4 * i + q, gbase + q)
        return carry

    lax.fori_loop(0, CHUNK // 4, block4, 0)

    for k in range(3):
        @pl.when(role == k)
        def _():
            pltpu.sync_copy(s_v, s_hbm.at[b, pl.ds(CHUNK * k, CHUNK)])
            pltpu.sync_copy(rowmax_v, rm_hbm.at[b, pl.ds(CHUNK * k, CHUNK)])

    @pl.when(role == 3)
    def _():
        pltpu.sync_copy(s_v.at[pl.ds(0, 32)], s_hbm.at[b, pl.ds(192, 32)])
        pltpu.sync_copy(rowmax_v.at[pl.ds(0, 32)],
                        rm_hbm.at[b, pl.ds(192, 32)])


@functools.partial(
    pl.kernel,
    out_type=jax.ShapeDtypeStruct((B, 32), jnp.int32),
    mesh=plsc.VectorSubcoreMesh(core_axis_name="c", subcore_axis_name="s"),
    scratch_types=[
        pltpu.VMEM((H, H), jnp.float32),
        pltpu.VMEM((H, H), jnp.float32),
        pltpu.VMEM((256,), jnp.float32),
        pltpu.VMEM((32,), jnp.int32),
        pltpu.VMEM((48,), jnp.float32),
    ],
)
def _greedy(hm_hbm, s_hbm, rm_hbm, out_hbm, hm_v, s_v, rowmax_v, coords_v,
            vtmp_v):
    cid = lax.axis_index("c")
    sid = lax.axis_index("s")
    wid = sid * 2 + cid
    iota = lax.iota(jnp.int32, 16)

    @pl.when(wid < B)
    def _():
        b = wid
        pltpu.sync_copy(hm_hbm.at[b], hm_v)
        pltpu.sync_copy(s_hbm.at[b], s_v)
        pltpu.sync_copy(rm_hbm.at[b], rowmax_v)
        rowmax_v[pl.ds(224, 16)] = jnp.full((16,), NEG, jnp.float32)
        rowmax_v[pl.ds(240, 16)] = jnp.full((16,), NEG, jnp.float32)

        def step(t, carry):
            macc = jnp.full((16,), NEG, jnp.float32)
            for j in range(14):
                macc = jnp.maximum(macc, rowmax_v[pl.ds(16 * j, 16)])
            gm = _xlane_max(macc)
            cand = jnp.full((16,), BIG, jnp.int32)
            for j in range(14):
                v = rowmax_v[pl.ds(16 * j, 16)]
                cand = jnp.minimum(cand, jnp.where(v == gm, iota + 16 * j, BIG))
            r_star = _xlane_min(cand)
            cand = jnp.full((16,), BIG, jnp.int32)
            for j in range(14):
                v = s_v[r_star, pl.ds(16 * j, 16)]
                cand = jnp.minimum(cand, jnp.where(v == gm, iota + 16 * j, BIG))
            c_star = _xlane_min(cand)
            base = pl.multiple_of((t // 8) * 16, 16)
            idx0 = 2 * t - base
            w = coords_v[pl.ds(base, 16)]
            w = jnp.where(iota == idx0, c_star + RAD, w)
            w = jnp.where(iota == idx0 + 1, r_star + RAD, w)
            coords_v[pl.ds(base, 16)] = w
            ca = pl.multiple_of((c_star // 16) * 16, 16)
            ca1 = pl.multiple_of(ca + 16, 16)
            in0 = (ca + iota >= c_star) & (ca + iota <= c_star + 6)
            in1 = (ca1 + iota >= c_star) & (ca1 + iota <= c_star + 6)
            for dr in range(7):
                row = r_star + dr
                w0 = hm_v[row, pl.ds(ca, 16)]
                hm_v[row, pl.ds(ca, 16)] = jnp.where(in0, jnp.float32(0.0), w0)
                w1 = hm_v[row, pl.ds(ca1, 16)]
                hm_v[row, pl.ds(ca1, 16)] = jnp.where(in1, jnp.float32(0.0), w1)
            cg = pl.multiple_of(jnp.minimum(
                (jnp.maximum(c_star - 6, 0) // 16) * 16, 192), 16)
            cg1 = pl.multiple_of(cg + 16, 16)
            cg2 = pl.multiple_of(cg + 32, 16)
            for d in range(13):
                rr = r_star - 6 + d

                @pl.when((rr >= 0) & (rr <= NR - 1))
                def _():
                    for half, off in enumerate((cg, cg1, cg2)):
                        v = _tree7([hm_v[rr + dr, pl.ds(off, 16)]
                                    for dr in range(7)])
                        vtmp_v[pl.ds(16 * half, 16)] = v
                    for half, off in enumerate((cg, cg1)):
                        snew = _tree7([vtmp_v[pl.ds(16 * half + dc, 16)]
                                       for dc in range(7)])
                        snew = jnp.where(off + iota <= 216, snew, NEG)
                        s_v[rr, pl.ds(off, 16)] = snew
                    macc2 = jnp.full((16,), NEG, jnp.float32)
                    for j in range(14):
                        macc2 = jnp.maximum(macc2, s_v[rr, pl.ds(16 * j, 16)])
                    _store_scalar(rowmax_v, rr, _xlane_max(macc2), iota)

            return carry

        lax.fori_loop(0, NT, step, 0)
        pltpu.sync_copy(coords_v, out_hbm.at[b])


def kernel(heatmap):
    hm3 = heatmap.reshape(B, H, H)
    s_all, rm_all = _build(hm3)
    out = _greedy(hm3, s_all, rm_all)
    return out.reshape(B, 16, 2)[:, :NT, :].astype(jnp.int64)


# speedup vs baseline: 6.8511x; 1.0186x over previous
import functools

import jax
import jax.numpy as jnp
from jax import lax
from jax.experimental import pallas as pl
from jax.experimental.pallas import tpu as pltpu
from jax.experimental.pallas import tpu_sc as plsc

B = 8
H = 224
NR = 217
RAD = 3
NT = 10
NEG = -1e30
BIG = 9999
CHUNK = 64


def _tree7(xs):
    a = (xs[0] + xs[1]) + (xs[2] + xs[3])
    b = (xs[4] + xs[5]) + xs[6]
    return a + b


def _xlane_max(v):
    w = jnp.maximum(v, lax.rev(v, (0,)))
    m = w[0]
    for l in range(1, 8):
        m = jnp.maximum(m, w[l])
    return m


def _xlane_min(v):
    w = jnp.minimum(v, lax.rev(v, (0,)))
    m = w[0]
    for l in range(1, 8):
        m = jnp.minimum(m, w[l])
    return m


def _store_scalar(ref, i, val, iota):
    base = pl.multiple_of((i // 16) * 16, 16)
    w = ref[pl.ds(base, 16)]
    ref[pl.ds(base, 16)] = jnp.where(base + iota == i, val, w)


@functools.partial(
    pl.kernel,
    out_type=(jax.ShapeDtypeStruct((B, H, H), jnp.float32),
              jax.ShapeDtypeStruct((B, 256), jnp.float32)),
    mesh=plsc.VectorSubcoreMesh(core_axis_name="c", subcore_axis_name="s"),
    scratch_types=[
        pltpu.VMEM((232, H), jnp.float32),
        pltpu.VMEM((CHUNK, H), jnp.float32),
        pltpu.VMEM((4, 240), jnp.float32),
        pltpu.VMEM((CHUNK,), jnp.float32),
    ],
)
def _build(hm_hbm, s_hbm, rm_hbm, hm_v, s_v, vrow_v, rowmax_v):
    cid = lax.axis_index("c")
    sid = lax.axis_index("s")
    img = sid // 4
    role = sid % 4
    b = cid * 4 + img
    iota = lax.iota(jnp.int32, 16)
    r0 = CHUNK * role

    for k in range(3):
        @pl.when(role == k)
        def _():
            pltpu.sync_copy(hm_hbm.at[b, pl.ds(CHUNK * k, 72)],
                            hm_v.at[pl.ds(CHUNK * k, 72)])

    @pl.when(role == 3)
    def _():
        pltpu.sync_copy(hm_hbm.at[b, pl.ds(192, 32)],
                        hm_v.at[pl.ds(192, 32)])

    for q in range(4):
        vrow_v[q, pl.ds(224, 16)] = jnp.zeros((16,), jnp.float32)

    def _horizontal(q, lr, gr):
        macc = jnp.full((16,), NEG, jnp.float32)
        for j in range(14):
            sj = _tree7([vrow_v[q, pl.ds(16 * j + dc, 16)]
                         for dc in range(7)])
            if j == 13:
                sj = jnp.where(iota + 208 <= 216, sj, NEG)
            s_v[lr, pl.ds(16 * j, 16)] = sj
            macc = jnp.maximum(macc, sj)
        val = jnp.where(gr <= NR - 1, _xlane_max(macc), NEG)
        _store_scalar(rowmax_v, lr, val, iota)

    def block4(i, carry):
        gbase = r0 + 4 * i

        @pl.when(gbase < H)
        def _():
            for j in range(14):
                h = [hm_v[gbase + dr, pl.ds(16 * j, 16)] for dr in range(10)]
                for q in range(4):
                    vrow_v[q, pl.ds(16 * j, 16)] = _tree7(h[q:q + 7])
            for q in range(4):
                _horizontal(q, 4 * i + q, gbase + q)
        return carry

    lax.fori_loop(0, CHUNK // 4, block4, 0)

    for k in range(3):
        @pl.when(role == k)
        def _():
            pltpu.sync_copy(s_v, s_hbm.at[b, pl.ds(CHUNK * k, CHUNK)])
            pltpu.sync_copy(rowmax_v, rm_hbm.at[b, pl.ds(CHUNK * k, CHUNK)])

    @pl.when(role == 3)
    def _():
        pltpu.sync_copy(s_v.at[pl.ds(0, 32)], s_hbm.at[b, pl.ds(192, 32)])
        pltpu.sync_copy(rowmax_v.at[pl.ds(0, 32)],
                        rm_hbm.at[b, pl.ds(192, 32)])


@functools.partial(
    pl.kernel,
    out_type=jax.ShapeDtypeStruct((B, 32), jnp.int32),
    mesh=plsc.VectorSubcoreMesh(core_axis_name="c", subcore_axis_name="s"),
    scratch_types=[
        pltpu.VMEM((H, H), jnp.float32),
        pltpu.VMEM((H, H), jnp.float32),
        pltpu.VMEM((256,), jnp.float32),
        pltpu.VMEM((32,), jnp.int32),
        pltpu.VMEM((48,), jnp.float32),
    ],
)
def _greedy(hm_hbm, s_hbm, rm_hbm, out_hbm, hm_v, s_v, rowmax_v, coords_v,
            vtmp_v):
    cid = lax.axis_index("c")
    sid = lax.axis_index("s")
    wid = sid * 2 + cid
    iota = lax.iota(jnp.int32, 16)

    @pl.when(wid < B)
    def _():
        b = wid
        pltpu.sync_copy(hm_hbm.at[b], hm_v)
        pltpu.sync_copy(s_hbm.at[b], s_v)
        pltpu.sync_copy(rm_hbm.at[b], rowmax_v)
        rowmax_v[pl.ds(224, 16)] = jnp.full((16,), NEG, jnp.float32)
        rowmax_v[pl.ds(240, 16)] = jnp.full((16,), NEG, jnp.float32)

        def step(t, carry):
            macc = jnp.full((16,), NEG, jnp.float32)
            for j in range(14):
                macc = jnp.maximum(macc, rowmax_v[pl.ds(16 * j, 16)])
            gm = _xlane_max(macc)
            cand = jnp.full((16,), BIG, jnp.int32)
            for j in range(14):
                v = rowmax_v[pl.ds(16 * j, 16)]
                cand = jnp.minimum(cand, jnp.where(v == gm, iota + 16 * j, BIG))
            r_star = _xlane_min(cand)
            cand = jnp.full((16,), BIG, jnp.int32)
            for j in range(14):
                v = s_v[r_star, pl.ds(16 * j, 16)]
                cand = jnp.minimum(cand, jnp.where(v == gm, iota + 16 * j, BIG))
            c_star = _xlane_min(cand)
            base = pl.multiple_of((t // 8) * 16, 16)
            idx0 = 2 * t - base
            w = coords_v[pl.ds(base, 16)]
            w = jnp.where(iota == idx0, c_star + RAD, w)
            w = jnp.where(iota == idx0 + 1, r_star + RAD, w)
            coords_v[pl.ds(base, 16)] = w
            ca = pl.multiple_of((c_star // 16) * 16, 16)
            ca1 = pl.multiple_of(ca + 16, 16)
            in0 = (ca + iota >= c_star) & (ca + iota <= c_star + 6)
            in1 = (ca1 + iota >= c_star) & (ca1 + iota <= c_star + 6)
            for dr in range(7):
                row = r_star + dr
                w0 = hm_v[row, pl.ds(ca, 16)]
                hm_v[row, pl.ds(ca, 16)] = jnp.where(in0, jnp.float32(0.0), w0)
                w1 = hm_v[row, pl.ds(ca1, 16)]
                hm_v[row, pl.ds(ca1, 16)] = jnp.where(in1, jnp.float32(0.0), w1)
            cg = pl.multiple_of(jnp.minimum(
                (jnp.maximum(c_star - 6, 0) // 16) * 16, 192), 16)
            cg1 = pl.multiple_of(cg + 16, 16)
            cg2 = pl.multiple_of(cg + 32, 16)
            for d in range(13):
                rr = jnp.minimum(jnp.maximum(r_star - 6 + d, 0), NR - 1)
                for half, off in enumerate((cg, cg1, cg2)):
                    v = _tree7([hm_v[rr + dr, pl.ds(off, 16)]
                                for dr in range(7)])
                    vtmp_v[pl.ds(16 * half, 16)] = v
                for half, off in enumerate((cg, cg1)):
                    snew = _tree7([vtmp_v[pl.ds(16 * half + dc, 16)]
                                   for dc in range(7)])
                    snew = jnp.where(off + iota <= 216, snew, NEG)
                    s_v[rr, pl.ds(off, 16)] = snew
                macc2 = jnp.full((16,), NEG, jnp.float32)
                for j in range(14):
                    macc2 = jnp.maximum(macc2, s_v[rr, pl.ds(16 * j, 16)])
                _store_scalar(rowmax_v, rr, _xlane_max(macc2), iota)

            return carry

        lax.fori_loop(0, NT, step, 0)
        pltpu.sync_copy(coords_v, out_hbm.at[b])


def kernel(heatmap):
    hm3 = heatmap.reshape(B, H, H)
    s_all, rm_all = _build(hm3)
    out = _greedy(hm3, s_all, rm_all)
    return out.reshape(B, 16, 2)[:, :NT, :].astype(jnp.int64)


# speedup vs baseline: 7.3382x; 1.0711x over previous
import functools

import jax
import jax.numpy as jnp
from jax import lax
from jax.experimental import pallas as pl
from jax.experimental.pallas import tpu as pltpu
from jax.experimental.pallas import tpu_sc as plsc

B = 8
H = 224
NR = 217
RAD = 3
NT = 10
NEG = -1e30
BIG = 9999
CHUNK = 64


def _tree7(xs):
    a = (xs[0] + xs[1]) + (xs[2] + xs[3])
    b = (xs[4] + xs[5]) + xs[6]
    return a + b


def _xlane_max(v):
    w = jnp.maximum(v, lax.rev(v, (0,)))
    m = w[0]
    for l in range(1, 8):
        m = jnp.maximum(m, w[l])
    return m


def _xlane_min(v):
    w = jnp.minimum(v, lax.rev(v, (0,)))
    m = w[0]
    for l in range(1, 8):
        m = jnp.minimum(m, w[l])
    return m


def _store_scalar(ref, i, val, iota):
    base = pl.multiple_of((i // 16) * 16, 16)
    w = ref[pl.ds(base, 16)]
    ref[pl.ds(base, 16)] = jnp.where(base + iota == i, val, w)


@functools.partial(
    pl.kernel,
    out_type=(jax.ShapeDtypeStruct((B, H, H), jnp.float32),
              jax.ShapeDtypeStruct((B, 256), jnp.float32)),
    mesh=plsc.VectorSubcoreMesh(core_axis_name="c", subcore_axis_name="s"),
    scratch_types=[
        pltpu.VMEM((232, H), jnp.float32),
        pltpu.VMEM((CHUNK, H), jnp.float32),
        pltpu.VMEM((4, 240), jnp.float32),
        pltpu.VMEM((CHUNK,), jnp.float32),
    ],
)
def _build(hm_hbm, s_hbm, rm_hbm, hm_v, s_v, vrow_v, rowmax_v):
    cid = lax.axis_index("c")
    sid = lax.axis_index("s")
    img = sid // 4
    role = sid % 4
    b = cid * 4 + img
    iota = lax.iota(jnp.int32, 16)
    r0 = CHUNK * role

    for k in range(3):
        @pl.when(role == k)
        def _():
            pltpu.sync_copy(hm_hbm.at[b, pl.ds(CHUNK * k, 72)],
                            hm_v.at[pl.ds(CHUNK * k, 72)])

    @pl.when(role == 3)
    def _():
        pltpu.sync_copy(hm_hbm.at[b, pl.ds(192, 32)],
                        hm_v.at[pl.ds(192, 32)])

    for q in range(4):
        vrow_v[q, pl.ds(224, 16)] = jnp.zeros((16,), jnp.float32)

    def _horizontal(q, lr, gr):
        macc = jnp.full((16,), NEG, jnp.float32)
        for j in range(14):
            sj = _tree7([vrow_v[q, pl.ds(16 * j + dc, 16)]
                         for dc in range(7)])
            if j == 13:
                sj = jnp.where(iota + 208 <= 216, sj, NEG)
            s_v[lr, pl.ds(16 * j, 16)] = sj
            macc = jnp.maximum(macc, sj)
        val = jnp.where(gr <= NR - 1, _xlane_max(macc), NEG)
        _store_scalar(rowmax_v, lr, val, iota)

    def block4(i, carry):
        gbase = r0 + 4 * i

        @pl.when(gbase < H)
        def _():
            for j in range(14):
                h = [hm_v[gbase + dr, pl.ds(16 * j, 16)] for dr in range(10)]
                for q in range(4):
                    vrow_v[q, pl.ds(16 * j, 16)] = _tree7(h[q:q + 7])
            for q in range(4):
                _horizontal(q, 4 * i + q, gbase + q)
        return carry

    lax.fori_loop(0, CHUNK // 4, block4, 0)

    for k in range(3):
        @pl.when(role == k)
        def _():
            pltpu.sync_copy(s_v, s_hbm.at[b, pl.ds(CHUNK * k, CHUNK)])
            pltpu.sync_copy(rowmax_v, rm_hbm.at[b, pl.ds(CHUNK * k, CHUNK)])

    @pl.when(role == 3)
    def _():
        pltpu.sync_copy(s_v.at[pl.ds(0, 32)], s_hbm.at[b, pl.ds(192, 32)])
        pltpu.sync_copy(rowmax_v.at[pl.ds(0, 32)],
                        rm_hbm.at[b, pl.ds(192, 32)])


@functools.partial(
    pl.kernel,
    out_type=jax.ShapeDtypeStruct((B, 32), jnp.int32),
    mesh=plsc.VectorSubcoreMesh(core_axis_name="c", subcore_axis_name="s"),
    scratch_types=[
        pltpu.VMEM((H, H), jnp.float32),
        pltpu.VMEM((H, H), jnp.float32),
        pltpu.VMEM((256,), jnp.float32),
        pltpu.VMEM((32,), jnp.int32),
        pltpu.VMEM((13, 48), jnp.float32),
    ],
)
def _greedy(hm_hbm, s_hbm, rm_hbm, out_hbm, hm_v, s_v, rowmax_v, coords_v,
            vtmp_v):
    cid = lax.axis_index("c")
    sid = lax.axis_index("s")
    wid = sid * 2 + cid
    iota = lax.iota(jnp.int32, 16)

    @pl.when(wid < B)
    def _():
        b = wid
        pltpu.sync_copy(hm_hbm.at[b], hm_v)
        pltpu.sync_copy(s_hbm.at[b], s_v)
        pltpu.sync_copy(rm_hbm.at[b], rowmax_v)
        rowmax_v[pl.ds(224, 16)] = jnp.full((16,), NEG, jnp.float32)
        rowmax_v[pl.ds(240, 16)] = jnp.full((16,), NEG, jnp.float32)

        def step(t, carry):
            macc = jnp.full((16,), NEG, jnp.float32)
            for j in range(14):
                macc = jnp.maximum(macc, rowmax_v[pl.ds(16 * j, 16)])
            gm = _xlane_max(macc)
            cand = jnp.full((16,), BIG, jnp.int32)
            for j in range(14):
                v = rowmax_v[pl.ds(16 * j, 16)]
                cand = jnp.minimum(cand, jnp.where(v == gm, iota + 16 * j, BIG))
            r_star = _xlane_min(cand)
            cand = jnp.full((16,), BIG, jnp.int32)
            for j in range(14):
                v = s_v[r_star, pl.ds(16 * j, 16)]
                cand = jnp.minimum(cand, jnp.where(v == gm, iota + 16 * j, BIG))
            c_star = _xlane_min(cand)
            base = pl.multiple_of((t // 8) * 16, 16)
            idx0 = 2 * t - base
            w = coords_v[pl.ds(base, 16)]
            w = jnp.where(iota == idx0, c_star + RAD, w)
            w = jnp.where(iota == idx0 + 1, r_star + RAD, w)
            coords_v[pl.ds(base, 16)] = w
            ca = pl.multiple_of((c_star // 16) * 16, 16)
            ca1 = pl.multiple_of(ca + 16, 16)
            in0 = (ca + iota >= c_star) & (ca + iota <= c_star + 6)
            in1 = (ca1 + iota >= c_star) & (ca1 + iota <= c_star + 6)
            for dr in range(7):
                row = r_star + dr
                w0 = hm_v[row, pl.ds(ca, 16)]
                hm_v[row, pl.ds(ca, 16)] = jnp.where(in0, jnp.float32(0.0), w0)
                w1 = hm_v[row, pl.ds(ca1, 16)]
                hm_v[row, pl.ds(ca1, 16)] = jnp.where(in1, jnp.float32(0.0), w1)
            cg = pl.multiple_of(jnp.minimum(
                (jnp.maximum(c_star - 6, 0) // 16) * 16, 192), 16)
            cg1 = pl.multiple_of(cg + 16, 16)
            cg2 = pl.multiple_of(cg + 32, 16)
            rlo = jnp.minimum(jnp.maximum(r_star - 6, 0), NR - 13)
            for half, off in enumerate((cg, cg1, cg2)):
                hh = [hm_v[rlo + dr, pl.ds(off, 16)] for dr in range(19)]
                for d in range(13):
                    vtmp_v[d, pl.ds(16 * half, 16)] = _tree7(hh[d:d + 7])
            for d in range(13):
                rr = rlo + d
                for half, off in enumerate((cg, cg1)):
                    snew = _tree7([vtmp_v[d, pl.ds(16 * half + dc, 16)]
                                   for dc in range(7)])
                    snew = jnp.where(off + iota <= 216, snew, NEG)
                    s_v[rr, pl.ds(off, 16)] = snew
                macc2 = jnp.full((16,), NEG, jnp.float32)
                for j in range(14):
                    macc2 = jnp.maximum(macc2, s_v[rr, pl.ds(16 * j, 16)])
                _store_scalar(rowmax_v, rr, _xlane_max(macc2), iota)

            return carry

        lax.fori_loop(0, NT, step, 0)
        pltpu.sync_copy(coords_v, out_hbm.at[b])


def kernel(heatmap):
    hm3 = heatmap.reshape(B, H, H)
    s_all, rm_all = _build(hm3)
    out = _greedy(hm3, s_all, rm_all)
    return out.reshape(B, 16, 2)[:, :NT, :].astype(jnp.int64)
